# baseline (speedup 1.0000x reference)
import jax
import jax.numpy as jnp
from jax.experimental import pallas as pl

IMAGE_W = 512
IMAGE_H = 512
TOP_N = 1000
MIN_SCORE_THRESHOLD = 0.01
NMS_THRESHOLD = 0.6
MAX_DETECTION_NUM = 100


def _identity_kernel(x_ref, o_ref):
    o_ref[...] = x_ref[...]


def _pallas_identity(x):
    L, B = x.shape[0], x.shape[1]
    return pl.pallas_call(
        _identity_kernel,
        grid=(L, B),
        in_specs=[pl.BlockSpec((1, 1) + x.shape[2:], lambda l, b: (l, b) + (0,) * (x.ndim - 2))],
        out_specs=pl.BlockSpec((1, 1) + x.shape[2:], lambda l, b: (l, b) + (0,) * (x.ndim - 2)),
        out_shape=jax.ShapeDtypeStruct(x.shape, x.dtype),
    )(x)


def _snap(reg_preds, points_position):
    xy_min = points_position - reg_preds[:, 0:2]
    xy_max = points_position + reg_preds[:, 2:4]
    b = jnp.concatenate([xy_min, xy_max], axis=1).astype(jnp.int32)
    b = b.at[:, 0].set(jnp.maximum(b[:, 0], 0))
    b = b.at[:, 1].set(jnp.maximum(b[:, 1], 0))
    b = b.at[:, 2].set(jnp.minimum(b[:, 2], IMAGE_W - 1))
    b = b.at[:, 3].set(jnp.minimum(b[:, 3], IMAGE_H - 1))
    return b


def _nms_fixed(boxes, valid):
    n = boxes.shape[0]
    x1, y1, x2, y2 = boxes[:, 0], boxes[:, 1], boxes[:, 2], boxes[:, 3]
    areas = (x2 - x1) * (y2 - y1)
    idx = jnp.arange(n)

    def body(i, state):
        suppressed, keep = state
        kept = jnp.logical_not(suppressed[i])
        keep = keep.at[i].set(kept)
        xx1 = jnp.maximum(x1[i], x1)
        yy1 = jnp.maximum(y1[i], y1)
        xx2 = jnp.minimum(x2[i], x2)
        yy2 = jnp.minimum(y2[i], y2)
        w = jnp.maximum(0, xx2 - xx1)
        h = jnp.maximum(0, yy2 - yy1)
        inter = w * h
        union = areas[i] + areas - inter
        over = (union > 0) & (5 * inter > 3 * union)
        suppressed = suppressed | (kept & (idx > i) & over)
        return suppressed, keep

    suppressed0 = jnp.logical_not(valid)
    keep0 = jnp.zeros((n,), dtype=bool)
    _, keep = jax.lax.fori_loop(0, n, body, (suppressed0, keep0))
    return keep


def kernel(cls_heads, reg_heads, center_heads, batch_positions):
    cls_heads = _pallas_identity(cls_heads)
    L = cls_heads.shape[0]
    fs, fsc, fr, fc, fp = [], [], [], [], []
    for l in range(L):
        c = jax.nn.sigmoid(cls_heads[l])
        r = jnp.exp(reg_heads[l])
        ce = jax.nn.sigmoid(center_heads[l])
        B = c.shape[0]
        c = c.reshape(B, -1, c.shape[-1])
        r = r.reshape(B, -1, r.shape[-1])
        ce = ce.reshape(B, -1, ce.shape[-1])
        pos = batch_positions[l].reshape(B, -1, batch_positions.shape[-1])
        scores = jnp.max(c, axis=2)
        score_classes = jnp.argmax(c, axis=2)
        scores = jnp.sqrt(scores * ce.squeeze(-1))
        if scores.shape[1] >= TOP_N:
            scores, indexes = jax.lax.top_k(scores, TOP_N)
            score_classes = jnp.take_along_axis(score_classes, indexes, axis=1)
            r = jnp.take_along_axis(r, indexes[:, :, None], axis=1)
            ce = jnp.take_along_axis(ce, indexes[:, :, None], axis=1)
            pos = jnp.take_along_axis(pos, indexes[:, :, None], axis=1)
            c = jnp.take_along_axis(c, indexes[:, :, None], axis=1)
        fs.append(scores)
        fsc.append(score_classes)
        fr.append(r)
        fp.append(pos)
        fc.append(c)
    fs = jnp.concatenate(fs, axis=1)
    fsc = jnp.concatenate(fsc, axis=1)
    fr = jnp.concatenate(fr, axis=1)
    fp = jnp.concatenate(fp, axis=1)
    fc = jnp.concatenate(fc, axis=1)
    B = fs.shape[0]
    num_classes = fc.shape[-1]
    batch_scores, batch_classes, batch_bboxes, batch_cls = [], [], [], []
    for i in range(B):
        scores = fs[i]
        score_classes = fsc[i].astype(jnp.float32)
        pred_bboxes = _snap(fr[i], fp[i])
        cls_preds = fc[i]
        valid = scores > MIN_SCORE_THRESHOLD
        sort_key = jnp.where(valid, scores, -jnp.inf)
        order = jnp.argsort(-sort_key, stable=True)
        ss = scores[order]
        sc = score_classes[order]
        sb = pred_bboxes[order]
        sp = cls_preds[order]
        sv = valid[order]
        keep = _nms_fixed(sb, sv)
        rank = jnp.cumsum(keep.astype(jnp.int32)) - 1
        slot = jnp.where(keep & (rank < MAX_DETECTION_NUM), rank, MAX_DETECTION_NUM)
        o_s = (-jnp.ones((MAX_DETECTION_NUM,), dtype=jnp.float32)).at[slot].set(ss, mode="drop")
        o_c = (-jnp.ones((MAX_DETECTION_NUM,), dtype=jnp.float32)).at[slot].set(sc, mode="drop")
        o_b = (-jnp.ones((MAX_DETECTION_NUM, 4), dtype=jnp.float32)).at[slot].set(
            sb.astype(jnp.float32), mode="drop")
        o_p = (-jnp.ones((MAX_DETECTION_NUM, num_classes), dtype=jnp.float32)).at[slot].set(
            sp, mode="drop")
        batch_scores.append(o_s)
        batch_classes.append(o_c)
        batch_bboxes.append(o_b)
        batch_cls.append(o_p)
    return (
        jnp.stack(batch_scores, axis=0),
        jnp.stack(batch_classes, axis=0),
        jnp.stack(batch_bboxes, axis=0),
        jnp.stack(batch_cls, axis=0),
    )


# speedup vs baseline: 90.4530x; 90.4530x over previous
import functools

import jax
import jax.numpy as jnp
from jax import lax
from jax.experimental import pallas as pl
from jax.experimental.pallas import tpu as pltpu
from jax.experimental.pallas import tpu_sc as plsc

L = 5
B = 8
HW = 64 * 64
C = 80
LB = L * B
TOP_N = 1000
NCAND = 5120
CHUNK = 256
NCHUNK = NCAND // CHUNK
MAXDET = 100
OUT = 128
MIN_SCORE = 0.01
IMG_MAX = 511.0

_SC_NC = 2
_SC_NS = 16
_SC_NW = _SC_NC * _SC_NS


def _dense_body(cls_ref, reg_ref, ctr_ref, pos_ref,
                s_ref, c_ref, x1_ref, y1_ref, x2_ref, y2_ref):
    x = cls_ref[0, 0]
    sig = jax.nn.sigmoid(x)
    m = jnp.max(sig, axis=1, keepdims=True)
    ci = lax.broadcasted_iota(jnp.int32, (HW, C), 1)
    am = jnp.min(jnp.where(sig == m, ci, C), axis=1, keepdims=True)
    ce = jax.nn.sigmoid(ctr_ref[0, 0])
    s = jnp.sqrt(m * ce)
    r = jnp.exp(reg_ref[0, 0])
    p = pos_ref[0, 0]
    x1 = jnp.maximum((p[:, 0:1] - r[:, 0:1]).astype(jnp.int32), 0)
    y1 = jnp.maximum((p[:, 1:2] - r[:, 1:2]).astype(jnp.int32), 0)
    x2 = jnp.minimum((p[:, 0:1] + r[:, 2:3]).astype(jnp.int32), 511)
    y2 = jnp.minimum((p[:, 1:2] + r[:, 3:4]).astype(jnp.int32), 511)
    s_ref[0, 0] = s
    c_ref[0, 0] = am.astype(jnp.float32)
    x1_ref[0, 0] = x1.astype(jnp.float32)
    y1_ref[0, 0] = y1.astype(jnp.float32)
    x2_ref[0, 0] = x2.astype(jnp.float32)
    y2_ref[0, 0] = y2.astype(jnp.float32)


def _dense(cls4, reg4, ctr4, pos4):
    out = jax.ShapeDtypeStruct((L, B, HW, 1), jnp.float32)
    return pl.pallas_call(
        _dense_body,
        grid=(L, B),
        in_specs=[
            pl.BlockSpec((1, 1, HW, C), lambda l, b: (l, b, 0, 0)),
            pl.BlockSpec((1, 1, HW, 4), lambda l, b: (l, b, 0, 0)),
            pl.BlockSpec((1, 1, HW, 1), lambda l, b: (l, b, 0, 0)),
            pl.BlockSpec((1, 1, HW, 2), lambda l, b: (l, b, 0, 0)),
        ],
        out_specs=[pl.BlockSpec((1, 1, HW, 1), lambda l, b: (l, b, 0, 0))] * 6,
        out_shape=[out] * 6,
    )(cls4, reg4, ctr4, pos4)


def _thresh_body(s_ref, t_ref, p_ref):
    s = s_ref[...]
    k = lax.bitcast_convert_type(s, jnp.int32)
    posi = lax.broadcasted_iota(jnp.int32, (LB, HW), 1)

    def bs_body(_, st):
        lo, hi = st
        mid = (lo + hi) >> 1
        cnt = jnp.sum((k >= mid).astype(jnp.int32), axis=1, keepdims=True)
        ge = cnt >= TOP_N
        return (jnp.where(ge, mid, lo), jnp.where(ge, hi, mid))

    lo0 = jnp.zeros((LB, 1), jnp.int32)
    hi0 = jnp.full((LB, 1), 1 << 30, jnp.int32)
    lo, _ = lax.fori_loop(0, 31, bs_body, (lo0, hi0))
    t = lo
    q = jnp.sum((k > t).astype(jnp.int32), axis=1, keepdims=True)
    need = TOP_N - q
    need1 = jnp.maximum(need, 1)
    eq = k == t

    def ps_body(_, st):
        lo2, hi2 = st
        mid = lo2 + ((hi2 - lo2) >> 1)
        cnt = jnp.sum((eq & (posi <= mid)).astype(jnp.int32), axis=1, keepdims=True)
        ge = cnt >= need1
        return (jnp.where(ge, lo2, mid), jnp.where(ge, mid, hi2))

    l0 = jnp.full((LB, 1), -1, jnp.int32)
    h0 = jnp.full((LB, 1), HW - 1, jnp.int32)
    _, hi2 = lax.fori_loop(0, 13, ps_body, (l0, h0))
    p = jnp.where(need > 0, hi2, -1)
    tf = lax.bitcast_convert_type(t, jnp.float32)
    t_ref[...] = jnp.broadcast_to(tf, (LB, 128))
    p_ref[...] = jnp.broadcast_to(p, (LB, 128))


def _thresh(s40):
    outf = jax.ShapeDtypeStruct((LB, 128), jnp.float32)
    outi = jax.ShapeDtypeStruct((LB, 128), jnp.int32)
    return pl.pallas_call(_thresh_body, out_shape=[outf, outi])(s40)


def _compact_body(score_h, class_h, x1_h, y1_h, x2_h, y2_h, t_h, p_h,
                  key_o, id_o, cls_o, bx1_o, by1_o, bx2_o, by2_o,
                  s_v, c_v, x1_v, y1_v, x2_v, y2_v, t_row, p_row,
                  ck, cid, ccl, cx1, cy1, cx2, cy2):
    w = lax.axis_index("s") * _SC_NC + lax.axis_index("c")

    def do_unit(u):
        l = u // B
        b = u - l * B
        pltpu.sync_copy(score_h.at[pl.ds(u * HW, HW)], s_v)
        pltpu.sync_copy(class_h.at[pl.ds(u * HW, HW)], c_v)
        pltpu.sync_copy(x1_h.at[pl.ds(u * HW, HW)], x1_v)
        pltpu.sync_copy(y1_h.at[pl.ds(u * HW, HW)], y1_v)
        pltpu.sync_copy(x2_h.at[pl.ds(u * HW, HW)], x2_v)
        pltpu.sync_copy(y2_h.at[pl.ds(u * HW, HW)], y2_v)
        pltpu.sync_copy(t_h.at[pl.ds(u * 128, 128)], t_row)
        pltpu.sync_copy(p_h.at[pl.ds(u * 128, 128)], p_row)

        @pl.when(l == 0)
        def _():
            neg1 = jnp.full((16,), -1.0, jnp.float32)
            z = jnp.zeros((16,), jnp.float32)
            for i in range(8):
                sl = pl.ds(i * 16, 16)
                ck[sl] = neg1
                cid[sl] = z
                ccl[sl] = z
                cx1[sl] = z
                cy1[sl] = z
                cx2[sl] = z
                cy2[sl] = z
            for buf, oarr in ((ck, key_o), (cid, id_o), (ccl, cls_o),
                              (cx1, bx1_o), (cy1, by1_o), (cx2, bx2_o),
                              (cy2, by2_o)):
                pltpu.sync_copy(buf.at[pl.ds(0, 120)],
                                oarr.at[pl.ds(b * NCAND + L * TOP_N, 120)])

        tv = t_row[pl.ds(0, 16)]
        pv = p_row[pl.ds(0, 16)]
        lane = lax.iota(jnp.int32, 16)

        def body(i, off):
            sl = pl.ds(i * 16, 16)
            s = s_v[sl]
            pos = i * 16 + lane
            qual = (s > tv) | ((s == tv) & (pos <= pv))
            key = jnp.where(s > MIN_SCORE, s, -1.0)
            idv = (l * HW + pos).astype(jnp.float32)
            plsc.store_compressed(ck.at[pl.ds(off, 16)], key, mask=qual)
            plsc.store_compressed(cid.at[pl.ds(off, 16)], idv, mask=qual)
            plsc.store_compressed(ccl.at[pl.ds(off, 16)], c_v[sl], mask=qual)
            plsc.store_compressed(cx1.at[pl.ds(off, 16)], x1_v[sl], mask=qual)
            plsc.store_compressed(cy1.at[pl.ds(off, 16)], y1_v[sl], mask=qual)
            plsc.store_compressed(cx2.at[pl.ds(off, 16)], x2_v[sl], mask=qual)
            plsc.store_compressed(cy2.at[pl.ds(off, 16)], y2_v[sl], mask=qual)
            return off + jnp.sum(qual.astype(jnp.int32))

        lax.fori_loop(0, HW // 16, body, jnp.int32(0))
        for buf, oarr in ((ck, key_o), (cid, id_o), (ccl, cls_o),
                          (cx1, bx1_o), (cy1, by1_o), (cx2, bx2_o),
                          (cy2, by2_o)):
            pltpu.sync_copy(buf.at[pl.ds(0, TOP_N)],
                            oarr.at[pl.ds(b * NCAND + l * TOP_N, TOP_N)])

    do_unit(w)

    @pl.when(w < LB - _SC_NW)
    def _():
        do_unit(w + _SC_NW)


def _compact(s40, c40, x140, y140, x240, y240, t40, p40):
    out = jax.ShapeDtypeStruct((B * NCAND,), jnp.float32)
    f = functools.partial(
        pl.kernel,
        out_type=[out] * 7,
        mesh=plsc.VectorSubcoreMesh(core_axis_name="c", subcore_axis_name="s"),
        compiler_params=pltpu.CompilerParams(needs_layout_passes=False),
        scratch_types=(
            [pltpu.VMEM((HW,), jnp.float32)] * 6
            + [pltpu.VMEM((128,), jnp.float32), pltpu.VMEM((128,), jnp.int32)]
            + [pltpu.VMEM((1024,), jnp.float32)] * 7
        ),
    )(_compact_body)
    return f(s40, c40, x140, y140, x240, y240, t40, p40)


_IB = 512
_JB = 1024


def _rank_body(ki_ref, kj_ref, o_ref):
    ib = pl.program_id(1)
    jb = pl.program_id(2)
    ki = ki_ref[0]
    kj = kj_ref[0]
    ii = ib * _IB + lax.broadcasted_iota(jnp.int32, (_IB, _JB), 0)
    jj = jb * _JB + lax.broadcasted_iota(jnp.int32, (_IB, _JB), 1)
    before = (kj > ki) | ((kj == ki) & (jj < ii))
    part = jnp.sum(before.astype(jnp.float32), axis=1, keepdims=True)

    @pl.when(jb == 0)
    def _():
        o_ref[0] = part

    @pl.when(jb != 0)
    def _():
        o_ref[0] += part


def _rank(key_col, key_row):
    return pl.pallas_call(
        _rank_body,
        grid=(B, NCAND // _IB, NCAND // _JB),
        in_specs=[
            pl.BlockSpec((1, _IB, 1), lambda b, ib, jb: (b, ib, 0)),
            pl.BlockSpec((1, 1, _JB), lambda b, ib, jb: (b, 0, jb)),
        ],
        out_specs=pl.BlockSpec((1, _IB, 1), lambda b, ib, jb: (b, ib, 0)),
        out_shape=jax.ShapeDtypeStruct((B, NCAND, 1), jnp.float32),
    )(key_col, key_row)


def _iou_suppress(bx1, by1, bx2, by2, bar, sup_ref, area_ref,
                  x1_s, y1_s, x2_s, y2_s, cb, extra_mask_fn):
    for cv in range(CHUNK // 16):
        sl = pl.ds(cb + cv * 16, 16)
        xx1 = jnp.maximum(bx1, x1_s[sl])
        yy1 = jnp.maximum(by1, y1_s[sl])
        xx2 = jnp.minimum(bx2, x2_s[sl])
        yy2 = jnp.minimum(by2, y2_s[sl])
        inter = jnp.maximum(xx2 - xx1, 0.0) * jnp.maximum(yy2 - yy1, 0.0)
        union = bar + area_ref[sl] - inter
        over = (union > 0.0) & (5.0 * inter > 3.0 * union)
        if extra_mask_fn is not None:
            over = over & extra_mask_fn(cv)
        sup_ref[sl] = jnp.where(over, 1.0, sup_ref[sl])


def _nms_body(key_h, id_h, cls_h, x1_h, y1_h, x2_h, y2_h, rank_h,
              os_h, oc_h, ob_h, oid_h, oval_h,
              key_i, id_i, cls_i, x1_i, y1_i, x2_i, y2_i, rank_i,
              key_s, id_s, cls_s, x1_s, y1_s, x2_s, y2_s, area_s, sup, keepm,
              kx1, ky1, kx2, ky2, kar,
              os_b, oc_b, ox1_b, oy1_b, ox2_b, oy2_b, oid_b, oval_b):
    w = lax.axis_index("s") * _SC_NC + lax.axis_index("c")

    @pl.when(w < B)
    def _():
        b = w
        for src, dst in ((key_h, key_i), (id_h, id_i), (cls_h, cls_i),
                         (x1_h, x1_i), (y1_h, y1_i), (x2_h, x2_i),
                         (y2_h, y2_i), (rank_h, rank_i)):
            pltpu.sync_copy(src.at[pl.ds(b * NCAND, NCAND)], dst)

        lane = lax.iota(jnp.int32, 16)

        def scat(i, _):
            sl = pl.ds(i * 16, 16)
            rv = rank_i[sl].astype(jnp.int32)
            kv = key_i[sl]
            plsc.store_scatter(key_s, [rv], kv)
            plsc.store_scatter(id_s, [rv], id_i[sl])
            plsc.store_scatter(cls_s, [rv], cls_i[sl])
            plsc.store_scatter(x1_s, [rv], x1_i[sl])
            plsc.store_scatter(y1_s, [rv], y1_i[sl])
            plsc.store_scatter(x2_s, [rv], x2_i[sl])
            plsc.store_scatter(y2_s, [rv], y2_i[sl])
            plsc.store_scatter(sup, [rv],
                               jnp.where(kv > MIN_SCORE, 0.0, 1.0))
            keepm[sl] = jnp.zeros((16,), jnp.float32)
            return 0

        lax.fori_loop(0, NCAND // 16, scat, 0)

        def ar(i, _):
            sl = pl.ds(i * 16, 16)
            area_s[sl] = (x2_s[sl] - x1_s[sl]) * (y2_s[sl] - y1_s[sl])
            return 0

        lax.fori_loop(0, NCAND // 16, ar, 0)

        def chunk_cond(st):
            c, nk = st
            return (c < NCHUNK) & (nk < MAXDET)

        def chunk_body(st):
            c, nk = st
            cb = c * CHUNK

            def p1(j, _):
                jv = jnp.full((16,), j, jnp.int32)
                bx1 = plsc.load_gather(kx1, [jv])
                by1 = plsc.load_gather(ky1, [jv])
                bx2 = plsc.load_gather(kx2, [jv])
                by2 = plsc.load_gather(ky2, [jv])
                bar = plsc.load_gather(kar, [jv])
                _iou_suppress(bx1, by1, bx2, by2, bar, sup, area_s,
                              x1_s, y1_s, x2_s, y2_s, cb, None)
                return 0

            lax.fori_loop(0, nk, p1, 0)

            def p2_cond(st2):
                i, nk2 = st2
                return (i < CHUNK) & (nk2 < MAXDET)

            def p2_body(st2):
                i, nk2 = st2
                g = cb + i
                base = cb + (i // 16) * 16
                ln = i - (i // 16) * 16
                sv = sup[pl.ds(base, 16)]
                s_here = jnp.sum(jnp.where(lane == ln, sv, 0.0))

                def kept_branch(nk_in):
                    gv = jnp.full((16,), g, jnp.int32)
                    bx1 = plsc.load_gather(x1_s, [gv])
                    by1 = plsc.load_gather(y1_s, [gv])
                    bx2 = plsc.load_gather(x2_s, [gv])
                    by2 = plsc.load_gather(y2_s, [gv])
                    bar = plsc.load_gather(area_s, [gv])
                    nkv = jnp.full((16,), nk_in, jnp.int32)
                    lane0 = lane == 0
                    one16 = jnp.ones((16,), jnp.float32)
                    plsc.store_scatter(kx1, [nkv], bx1, mask=lane0)
                    plsc.store_scatter(ky1, [nkv], by1, mask=lane0)
                    plsc.store_scatter(kx2, [nkv], bx2, mask=lane0)
                    plsc.store_scatter(ky2, [nkv], by2, mask=lane0)
                    plsc.store_scatter(kar, [nkv], bar, mask=lane0)
                    plsc.store_scatter(keepm, [gv], one16, mask=lane0)
                    _iou_suppress(
                        bx1, by1, bx2, by2, bar, sup, area_s,
                        x1_s, y1_s, x2_s, y2_s, cb,
                        lambda cv: (cb + cv * 16 + lane) > g)
                    return nk_in + 1

                nk3 = lax.cond(s_here == 0.0, kept_branch,
                               lambda nk_in: nk_in, nk2)
                return (i + 1, nk3)

            _, nk_f = lax.while_loop(p2_cond, p2_body, (jnp.int32(0), nk))
            return (c + 1, nk_f)

        _, nk_final = lax.while_loop(chunk_cond, chunk_body,
                                     (jnp.int32(0), jnp.int32(0)))

        neg1 = jnp.full((16,), -1.0, jnp.float32)
        z16 = jnp.zeros((16,), jnp.float32)
        for i in range(OUT // 16):
            sl = pl.ds(i * 16, 16)
            os_b[sl] = neg1
            oc_b[sl] = neg1
            ox1_b[sl] = neg1
            oy1_b[sl] = neg1
            ox2_b[sl] = neg1
            oy2_b[sl] = neg1
            oid_b[sl] = z16

        def comp(i, off):
            sl = pl.ds(i * 16, 16)
            m = keepm[sl] > 0.0
            plsc.store_compressed(os_b.at[pl.ds(off, 16)], key_s[sl], mask=m)
            plsc.store_compressed(oc_b.at[pl.ds(off, 16)], cls_s[sl], mask=m)
            plsc.store_compressed(ox1_b.at[pl.ds(off, 16)], x1_s[sl], mask=m)
            plsc.store_compressed(oy1_b.at[pl.ds(off, 16)], y1_s[sl], mask=m)
            plsc.store_compressed(ox2_b.at[pl.ds(off, 16)], x2_s[sl], mask=m)
            plsc.store_compressed(oy2_b.at[pl.ds(off, 16)], y2_s[sl], mask=m)
            plsc.store_compressed(oid_b.at[pl.ds(off, 16)], id_s[sl], mask=m)
            return off + jnp.sum(m.astype(jnp.int32))

        lax.fori_loop(0, NCAND // 16, comp, jnp.int32(0))

        lane16 = lax.iota(jnp.int32, 16)
        for i in range(OUT // 16):
            sl = pl.ds(i * 16, 16)
            oval_b[sl] = jnp.where(i * 16 + lane16 < nk_final, 1.0, 0.0)

        pltpu.sync_copy(os_b, os_h.at[pl.ds(b * OUT, OUT)])
        pltpu.sync_copy(oc_b, oc_h.at[pl.ds(b * OUT, OUT)])
        pltpu.sync_copy(ox1_b, ob_h.at[pl.ds((b * 4 + 0) * OUT, OUT)])
        pltpu.sync_copy(oy1_b, ob_h.at[pl.ds((b * 4 + 1) * OUT, OUT)])
        pltpu.sync_copy(ox2_b, ob_h.at[pl.ds((b * 4 + 2) * OUT, OUT)])
        pltpu.sync_copy(oy2_b, ob_h.at[pl.ds((b * 4 + 3) * OUT, OUT)])
        pltpu.sync_copy(oid_b, oid_h.at[pl.ds(b * OUT, OUT)])
        pltpu.sync_copy(oval_b, oval_h.at[pl.ds(b * OUT, OUT)])


def _nms(key, cid, ccl, bx1, by1, bx2, by2, rank2):
    f = functools.partial(
        pl.kernel,
        out_type=[
            jax.ShapeDtypeStruct((B * OUT,), jnp.float32),
            jax.ShapeDtypeStruct((B * OUT,), jnp.float32),
            jax.ShapeDtypeStruct((B * 4 * OUT,), jnp.float32),
            jax.ShapeDtypeStruct((B * OUT,), jnp.float32),
            jax.ShapeDtypeStruct((B * OUT,), jnp.float32),
        ],
        mesh=plsc.VectorSubcoreMesh(core_axis_name="c", subcore_axis_name="s"),
        compiler_params=pltpu.CompilerParams(needs_layout_passes=False),
        scratch_types=(
            [pltpu.VMEM((NCAND,), jnp.float32)] * 8
            + [pltpu.VMEM((NCAND,), jnp.float32)] * 10
            + [pltpu.VMEM((OUT,), jnp.float32)] * 5
            + [pltpu.VMEM((OUT,), jnp.float32)] * 8
        ),
    )(_nms_body)
    return f(key, cid, ccl, bx1, by1, bx2, by2, rank2)


def _clsgather_body(cls_ref, id_ref, val_ref, o_ref):
    l = pl.program_id(1)
    ids = id_ref[0]
    tgt = (l * HW + lax.broadcasted_iota(jnp.int32, (OUT, HW), 1)).astype(
        jnp.float32)
    oh = (ids == tgt).astype(jnp.float32)
    part = jnp.dot(oh, cls_ref[0, 0], preferred_element_type=jnp.float32)

    @pl.when(l == 0)
    def _():
        o_ref[0] = part

    @pl.when((l > 0) & (l < L - 1))
    def _():
        o_ref[0] += part

    @pl.when(l == L - 1)
    def _():
        acc = o_ref[0] + part
        sig = jax.nn.sigmoid(acc)
        o_ref[0] = jnp.where(val_ref[0] > 0.0, sig, -1.0)


def _clsgather(cls4, oid, oval):
    return pl.pallas_call(
        _clsgather_body,
        grid=(B, L),
        in_specs=[
            pl.BlockSpec((1, 1, HW, C), lambda b, l: (l, b, 0, 0)),
            pl.BlockSpec((1, OUT, 1), lambda b, l: (b, 0, 0)),
            pl.BlockSpec((1, OUT, 1), lambda b, l: (b, 0, 0)),
        ],
        out_specs=pl.BlockSpec((1, OUT, C), lambda b, l: (b, 0, 0)),
        out_shape=jax.ShapeDtypeStruct((B, OUT, C), jnp.float32),
    )(cls4, oid, oval)


def kernel(cls_heads, reg_heads, center_heads, batch_positions):
    cls4 = cls_heads.reshape(L, B, HW, C)
    reg4 = reg_heads.reshape(L, B, HW, 4)
    ctr4 = center_heads.reshape(L, B, HW, 1)
    pos4 = batch_positions.reshape(L, B, HW, 2)
    s5, c5, x15, y15, x25, y25 = _dense(cls4, reg4, ctr4, pos4)
    s40 = s5.reshape(LB, HW)
    c40 = c5.reshape(LB, HW)
    x140 = x15.reshape(LB, HW)
    y140 = y15.reshape(LB, HW)
    x240 = x25.reshape(LB, HW)
    y240 = y25.reshape(LB, HW)
    t40, p40 = _thresh(s40)
    key, cid, ccl, bx1, by1, bx2, by2 = _compact(
        s40.reshape(-1), c40.reshape(-1), x140.reshape(-1), y140.reshape(-1),
        x240.reshape(-1), y240.reshape(-1), t40.reshape(-1), p40.reshape(-1))
    rank = _rank(key.reshape(B, NCAND, 1), key.reshape(B, 1, NCAND))
    o_s, o_c, o_b, oid, oval = _nms(key, cid, ccl, bx1, by1, bx2, by2,
                                    rank.reshape(B * NCAND))
    o_p = _clsgather(cls4, oid.reshape(B, OUT, 1), oval.reshape(B, OUT, 1))
    return (
        o_s.reshape(B, OUT)[:, :MAXDET],
        o_c.reshape(B, OUT)[:, :MAXDET],
        jnp.transpose(o_b.reshape(B, 4, OUT), (0, 2, 1))[:, :MAXDET, :],
        o_p[:, :MAXDET, :],
    )
